# baseline (speedup 1.0000x reference)
import dataclasses
import functools

import jax
import jax.numpy as jnp
from jax.experimental import pallas as pl
from jax.experimental.pallas import tpu as pltpu
from jax.experimental.pallas import tpu_sc as plsc

_N = 10000
_E = 320000
_EALL = _E + _N
_D = 128

_NC = 2
_NS = 16
_NW = _NC * _NS
_L = 16
_ET = 10320
_E_PAD = _ET * _NW
_N_PAD = 10016
_FPT = _D // _NW

_mesh = plsc.VectorSubcoreMesh(core_axis_name="c", subcore_axis_name="s")

_sc_params = pltpu.CompilerParams()
if "needs_layout_passes" in pltpu.CompilerParams.__dataclass_fields__:
    _sc_params = dataclasses.replace(_sc_params, needs_layout_passes=False)


def _dense1_body(x_ref, w_ref, asr_ref, adr_ref, xw_ref, as_ref, ad_ref, am_ref):
    xw = jnp.dot(x_ref[...], w_ref[...], preferred_element_type=jnp.float32)
    xw_ref[...] = xw
    asv = jnp.dot(xw, asr_ref[...], preferred_element_type=jnp.float32)
    adv = jnp.dot(xw, adr_ref[...], preferred_element_type=jnp.float32)
    as_ref[...] = asv
    ad_ref[...] = adv
    am_ref[...] = jnp.max(asv).reshape(1, 1)


def _dense1(x, w, a_src, a_dst):
    return pl.pallas_call(
        _dense1_body,
        out_shape=(
            jax.ShapeDtypeStruct((_N, _D), jnp.float32),
            jax.ShapeDtypeStruct((_N, 1), jnp.float32),
            jax.ShapeDtypeStruct((_N, 1), jnp.float32),
            jax.ShapeDtypeStruct((1, 1), jnp.float32),
        ),
    )(x, w, a_src.reshape(_D, 1), a_dst.reshape(_D, 1))


def _dense2_body(hp_ref, b_ref, w_ref, asr_ref, adr_ref,
                 xw_ref, as_ref, ad_ref, am_ref):
    h = hp_ref[...] + b_ref[...]
    h = jnp.where(h > 0, h, jnp.exp(jnp.minimum(h, 0.0)) - 1.0)
    xw = jnp.dot(h, w_ref[...], preferred_element_type=jnp.float32)
    xw_ref[...] = xw
    asv = jnp.dot(xw, asr_ref[...], preferred_element_type=jnp.float32)
    adv = jnp.dot(xw, adr_ref[...], preferred_element_type=jnp.float32)
    as_ref[...] = asv
    ad_ref[...] = adv
    am_ref[...] = jnp.max(asv).reshape(1, 1)


def _dense2(hp, b, w, a_src, a_dst):
    return pl.pallas_call(
        _dense2_body,
        out_shape=(
            jax.ShapeDtypeStruct((_N, _D), jnp.float32),
            jax.ShapeDtypeStruct((_N, 1), jnp.float32),
            jax.ShapeDtypeStruct((_N, 1), jnp.float32),
            jax.ShapeDtypeStruct((1, 1), jnp.float32),
        ),
    )(hp, b.reshape(1, _D), w, a_src.reshape(_D, 1), a_dst.reshape(_D, 1))


def _rden_body(den_ref, r_ref):
    s = jnp.sum(den_ref[...], axis=0, keepdims=True)
    r_ref[...] = 1.0 / (s + 1e-16)


def _rden(den):
    return pl.pallas_call(
        _rden_body,
        out_shape=jax.ShapeDtypeStruct((1, _N_PAD), jnp.float32),
    )(den)


def _bias_body(u_ref, b_ref, o_ref):
    o_ref[...] = u_ref[...] + b_ref[...]


def _bias(u, b):
    return pl.pallas_call(
        _bias_body,
        out_shape=jax.ShapeDtypeStruct((_N, _D), jnp.float32),
    )(u, b.reshape(1, _D))


@functools.partial(
    pl.kernel,
    out_type=(
        jax.ShapeDtypeStruct((_E_PAD,), jnp.float32),
        jax.ShapeDtypeStruct((_NW, _N_PAD), jnp.float32),
    ),
    mesh=_mesh,
    compiler_params=_sc_params,
    scratch_types=[
        pltpu.VMEM((_N_PAD,), jnp.float32),
        pltpu.VMEM((_N_PAD,), jnp.float32),
        pltpu.VMEM((_L,), jnp.float32),
        pltpu.VMEM((_ET,), jnp.int32),
        pltpu.VMEM((_ET,), jnp.int32),
        pltpu.VMEM((_ET,), jnp.float32),
        pltpu.VMEM((_N_PAD,), jnp.float32),
    ],
)
def _edge_a(src_hbm, dst_hbm, as_hbm, ad_hbm, av_hbm, ee_hbm, den_hbm,
            asl, adl, avl, srcb, dstb, eeb, denl):
    wid = jax.lax.axis_index("s") * _NC + jax.lax.axis_index("c")
    base = wid * _ET
    pltpu.sync_copy(as_hbm, asl)
    pltpu.sync_copy(ad_hbm, adl)
    pltpu.sync_copy(av_hbm, avl)
    pltpu.sync_copy(src_hbm.at[pl.ds(base, _ET)], srcb)
    pltpu.sync_copy(dst_hbm.at[pl.ds(base, _ET)], dstb)

    @plsc.parallel_loop(0, _N_PAD, step=_L, unroll=8)
    def _(j):
        denl[pl.ds(j, _L)] = jnp.zeros((_L,), jnp.float32)

    av = avl[...]

    @plsc.parallel_loop(0, _ET, step=_L, unroll=4)
    def _(i):
        s16 = srcb[pl.ds(i, _L)]
        d16 = dstb[pl.ds(i, _L)]
        a = plsc.load_gather(asl, [s16])
        d = plsc.load_gather(adl, [d16])
        z = a + d
        e = jnp.where(z >= 0, z, 0.2 * z)
        zm = av + d
        m = jnp.where(zm >= 0, zm, 0.2 * zm)
        ee = jnp.exp(e - m)
        eeb[pl.ds(i, _L)] = ee
        plsc.addupdate_scatter(denl, [d16], ee)

    pltpu.sync_copy(eeb, ee_hbm.at[pl.ds(base, _ET)])
    pltpu.sync_copy(denl, den_hbm.at[wid])


@functools.partial(
    pl.kernel,
    out_type=jax.ShapeDtypeStruct((_E_PAD,), jnp.float32),
    mesh=_mesh,
    compiler_params=_sc_params,
    scratch_types=[
        pltpu.VMEM((_N_PAD,), jnp.float32),
        pltpu.VMEM((_ET,), jnp.int32),
        pltpu.VMEM((_ET,), jnp.float32),
    ],
)
def _edge_alpha(dst_hbm, ee_hbm, rd_hbm, al_hbm, rdl, dstb, eeb):
    wid = jax.lax.axis_index("s") * _NC + jax.lax.axis_index("c")
    base = wid * _ET
    pltpu.sync_copy(rd_hbm, rdl)
    pltpu.sync_copy(dst_hbm.at[pl.ds(base, _ET)], dstb)
    pltpu.sync_copy(ee_hbm.at[pl.ds(base, _ET)], eeb)

    @plsc.parallel_loop(0, _ET, step=_L, unroll=4)
    def _(i):
        d16 = dstb[pl.ds(i, _L)]
        r16 = plsc.load_gather(rdl, [d16])
        eeb[pl.ds(i, _L)] = eeb[pl.ds(i, _L)] * r16

    pltpu.sync_copy(eeb, al_hbm.at[pl.ds(base, _ET)])


_EC = 5504
_NCH = _E_PAD // _EC


@functools.partial(
    pl.kernel,
    out_type=jax.ShapeDtypeStruct((_D, _N_PAD), jnp.float32),
    mesh=_mesh,
    compiler_params=_sc_params,
    scratch_types=[
        [pltpu.VMEM((_N_PAD,), jnp.float32) for _ in range(_FPT)],
        [pltpu.VMEM((_N_PAD,), jnp.float32) for _ in range(_FPT)],
        [[pltpu.VMEM((_EC,), jnp.int32),
          pltpu.VMEM((_EC,), jnp.int32),
          pltpu.VMEM((_EC,), jnp.float32)] for _ in range(2)],
    ],
)
def _edge_b(src_hbm, dst_hbm, al_hbm, xwt_hbm, ut_hbm, xws, uts, bufs):
    wid = jax.lax.axis_index("s") * _NC + jax.lax.axis_index("c")
    f0 = wid * _FPT
    for f in range(_FPT):
        pltpu.sync_copy(xwt_hbm.at[f0 + f], xws[f])

        @plsc.parallel_loop(0, _N_PAD, step=_L, unroll=8)
        def _(j, _u=uts[f]):
            _u[pl.ds(j, _L)] = jnp.zeros((_L,), jnp.float32)

    def start(k, b):
        srcb, dstb, alb = bufs[b]
        cb = k * _EC
        pltpu.sync_copy(src_hbm.at[pl.ds(cb, _EC)], srcb)
        pltpu.sync_copy(dst_hbm.at[pl.ds(cb, _EC)], dstb)
        pltpu.sync_copy(al_hbm.at[pl.ds(cb, _EC)], alb)

    def compute(b):
        srcb, dstb, alb = bufs[b]

        @plsc.parallel_loop(0, _EC, step=_L, unroll=8)
        def _(i):
            s16 = srcb[pl.ds(i, _L)]
            d16 = dstb[pl.ds(i, _L)]
            a16 = alb[pl.ds(i, _L)]
            for f in range(_FPT):
                g = plsc.load_gather(xws[f], [s16])
                plsc.addupdate_scatter(uts[f], [d16], a16 * g)

    @pl.loop(0, _NCH, step=2)
    def _(k):
        start(k, 0)
        compute(0)
        start(k + 1, 1)
        compute(1)

    for f in range(_FPT):
        pltpu.sync_copy(uts[f], ut_hbm.at[f0 + f])


def _layer(src, dst, xw, asv, adv, amax):
    asp = jnp.concatenate(
        [asv[:, 0], jnp.full((_N_PAD - _N,), -1e30, jnp.float32)])
    adp = jnp.concatenate([adv[:, 0], jnp.zeros((_N_PAD - _N,), jnp.float32)])
    avec = jnp.broadcast_to(amax.reshape(1), (_L,))
    ee, den = _edge_a(src, dst, asp, adp, avec)
    rden = _rden(den)[0]
    alpha = _edge_alpha(dst, ee, rden)
    xwt = jnp.pad(xw.T, ((0, 0), (0, _N_PAD - _N)))
    ut = _edge_b(src, dst, alpha, xwt)
    return ut, alpha[:_EALL]


def kernel(x, edge_index, W1, a_src1, a_dst1, b1, W2, a_src2, a_dst2, b2):
    loop = jnp.arange(_N, dtype=edge_index.dtype)
    src0 = jnp.concatenate([edge_index[0], loop])
    dst0 = jnp.concatenate([edge_index[1], loop])
    ei = jnp.stack([src0, dst0], axis=0)
    padi = jnp.full((_E_PAD - _EALL,), _N, jnp.int32)
    src = jnp.concatenate([src0.astype(jnp.int32), padi])
    dst = jnp.concatenate([dst0.astype(jnp.int32), padi])

    xw1, as1, ad1, am1 = _dense1(x, W1, a_src1, a_dst1)
    ut1, alpha1 = _layer(src, dst, xw1, as1, ad1, am1)

    hpre = jnp.transpose(ut1)[:_N]
    xw2, as2, ad2, am2 = _dense2(hpre, b1, W2, a_src2, a_dst2)
    ut2, alpha2 = _layer(src, dst, xw2, as2, ad2, am2)

    out = _bias(jnp.transpose(ut2)[:_N], b2)
    return (out, ((ei, alpha1), (ei, alpha2)))


# speedup vs baseline: 1.5236x; 1.5236x over previous
import dataclasses
import functools

import jax
import jax.numpy as jnp
from jax.experimental import pallas as pl
from jax.experimental.pallas import tpu as pltpu
from jax.experimental.pallas import tpu_sc as plsc

_N = 10000
_E = 320000
_EALL = _E + _N
_D = 128

_NC = 2
_NS = 16
_NW = _NC * _NS
_L = 16
_ET = 10320
_E_PAD = _ET * _NW
_N_PAD = 10016
_FPT = _D // _NW

_mesh = plsc.VectorSubcoreMesh(core_axis_name="c", subcore_axis_name="s")

_sc_params = pltpu.CompilerParams()
if "needs_layout_passes" in pltpu.CompilerParams.__dataclass_fields__:
    _sc_params = dataclasses.replace(_sc_params, needs_layout_passes=False)


def _dense1_body(x_ref, w_ref, asr_ref, adr_ref, xw_ref, as_ref, ad_ref, am_ref):
    xw = jnp.dot(x_ref[...], w_ref[...], preferred_element_type=jnp.float32)
    xw_ref[...] = xw
    asv = jnp.dot(xw, asr_ref[...], preferred_element_type=jnp.float32)
    adv = jnp.dot(xw, adr_ref[...], preferred_element_type=jnp.float32)
    as_ref[...] = asv
    ad_ref[...] = adv
    am_ref[...] = jnp.max(asv).reshape(1, 1)


def _dense1(x, w, a_src, a_dst):
    return pl.pallas_call(
        _dense1_body,
        out_shape=(
            jax.ShapeDtypeStruct((_N, _D), jnp.float32),
            jax.ShapeDtypeStruct((_N, 1), jnp.float32),
            jax.ShapeDtypeStruct((_N, 1), jnp.float32),
            jax.ShapeDtypeStruct((1, 1), jnp.float32),
        ),
    )(x, w, a_src.reshape(_D, 1), a_dst.reshape(_D, 1))


def _dense2_body(hp_ref, b_ref, w_ref, asr_ref, adr_ref,
                 xw_ref, as_ref, ad_ref, am_ref):
    h = hp_ref[...] + b_ref[...]
    h = jnp.where(h > 0, h, jnp.exp(jnp.minimum(h, 0.0)) - 1.0)
    xw = jnp.dot(h, w_ref[...], preferred_element_type=jnp.float32)
    xw_ref[...] = xw
    asv = jnp.dot(xw, asr_ref[...], preferred_element_type=jnp.float32)
    adv = jnp.dot(xw, adr_ref[...], preferred_element_type=jnp.float32)
    as_ref[...] = asv
    ad_ref[...] = adv
    am_ref[...] = jnp.max(asv).reshape(1, 1)


def _dense2(hp, b, w, a_src, a_dst):
    return pl.pallas_call(
        _dense2_body,
        out_shape=(
            jax.ShapeDtypeStruct((_N, _D), jnp.float32),
            jax.ShapeDtypeStruct((_N, 1), jnp.float32),
            jax.ShapeDtypeStruct((_N, 1), jnp.float32),
            jax.ShapeDtypeStruct((1, 1), jnp.float32),
        ),
    )(hp, b.reshape(1, _D), w, a_src.reshape(_D, 1), a_dst.reshape(_D, 1))


def _rden_body(den_ref, r_ref):
    s = jnp.sum(den_ref[...], axis=0, keepdims=True)
    r_ref[...] = 1.0 / (s + 1e-16)


def _rden(den):
    return pl.pallas_call(
        _rden_body,
        out_shape=jax.ShapeDtypeStruct((1, _N_PAD), jnp.float32),
    )(den)


def _bias_body(u_ref, b_ref, o_ref):
    o_ref[...] = u_ref[...] + b_ref[...]


def _bias(u, b):
    return pl.pallas_call(
        _bias_body,
        out_shape=jax.ShapeDtypeStruct((_N, _D), jnp.float32),
    )(u, b.reshape(1, _D))


@functools.partial(
    pl.kernel,
    out_type=(
        jax.ShapeDtypeStruct((_E_PAD,), jnp.float32),
        jax.ShapeDtypeStruct((_NW, _N_PAD), jnp.float32),
    ),
    mesh=_mesh,
    compiler_params=_sc_params,
    scratch_types=[
        pltpu.VMEM((_N_PAD,), jnp.float32),
        pltpu.VMEM((_N_PAD,), jnp.float32),
        pltpu.VMEM((_L,), jnp.float32),
        pltpu.VMEM((_ET,), jnp.int32),
        pltpu.VMEM((_ET,), jnp.int32),
        pltpu.VMEM((_ET,), jnp.float32),
        pltpu.VMEM((_N_PAD,), jnp.float32),
    ],
)
def _edge_a(src_hbm, dst_hbm, as_hbm, ad_hbm, av_hbm, ee_hbm, den_hbm,
            asl, adl, avl, srcb, dstb, eeb, denl):
    wid = jax.lax.axis_index("s") * _NC + jax.lax.axis_index("c")
    base = wid * _ET
    pltpu.sync_copy(as_hbm, asl)
    pltpu.sync_copy(ad_hbm, adl)
    pltpu.sync_copy(av_hbm, avl)
    pltpu.sync_copy(src_hbm.at[pl.ds(base, _ET)], srcb)
    pltpu.sync_copy(dst_hbm.at[pl.ds(base, _ET)], dstb)

    @plsc.parallel_loop(0, _N_PAD, step=_L, unroll=8)
    def _(j):
        denl[pl.ds(j, _L)] = jnp.zeros((_L,), jnp.float32)

    av = avl[...]

    @plsc.parallel_loop(0, _ET, step=_L, unroll=4)
    def _(i):
        s16 = srcb[pl.ds(i, _L)]
        d16 = dstb[pl.ds(i, _L)]
        a = plsc.load_gather(asl, [s16])
        d = plsc.load_gather(adl, [d16])
        z = a + d
        e = jnp.where(z >= 0, z, 0.2 * z)
        zm = av + d
        m = jnp.where(zm >= 0, zm, 0.2 * zm)
        ee = jnp.exp(e - m)
        eeb[pl.ds(i, _L)] = ee
        plsc.addupdate_scatter(denl, [d16], ee)

    pltpu.sync_copy(eeb, ee_hbm.at[pl.ds(base, _ET)])
    pltpu.sync_copy(denl, den_hbm.at[wid])


@functools.partial(
    pl.kernel,
    out_type=jax.ShapeDtypeStruct((_E_PAD,), jnp.float32),
    mesh=_mesh,
    compiler_params=_sc_params,
    scratch_types=[
        pltpu.VMEM((_N_PAD,), jnp.float32),
        pltpu.VMEM((_ET,), jnp.int32),
        pltpu.VMEM((_ET,), jnp.float32),
    ],
)
def _edge_alpha(dst_hbm, ee_hbm, rd_hbm, al_hbm, rdl, dstb, eeb):
    wid = jax.lax.axis_index("s") * _NC + jax.lax.axis_index("c")
    base = wid * _ET
    pltpu.sync_copy(rd_hbm, rdl)
    pltpu.sync_copy(dst_hbm.at[pl.ds(base, _ET)], dstb)
    pltpu.sync_copy(ee_hbm.at[pl.ds(base, _ET)], eeb)

    @plsc.parallel_loop(0, _ET, step=_L, unroll=4)
    def _(i):
        d16 = dstb[pl.ds(i, _L)]
        r16 = plsc.load_gather(rdl, [d16])
        eeb[pl.ds(i, _L)] = eeb[pl.ds(i, _L)] * r16

    pltpu.sync_copy(eeb, al_hbm.at[pl.ds(base, _ET)])


_EC = 5504
_NCH = _E_PAD // _EC


@functools.partial(
    pl.kernel,
    out_type=jax.ShapeDtypeStruct((_D, _N_PAD), jnp.float32),
    mesh=_mesh,
    compiler_params=_sc_params,
    scratch_types=[
        [pltpu.VMEM((_N_PAD,), jnp.float32) for _ in range(_FPT)],
        [pltpu.VMEM((_N_PAD,), jnp.float32) for _ in range(_FPT)],
        [[pltpu.VMEM((_EC,), jnp.int32),
          pltpu.VMEM((_EC,), jnp.int32),
          pltpu.VMEM((_EC,), jnp.float32)] for _ in range(2)],
        [pltpu.SemaphoreType.DMA for _ in range(2)],
    ],
)
def _edge_b(src_hbm, dst_hbm, al_hbm, xwt_hbm, ut_hbm, xws, uts, bufs, sems):
    wid = jax.lax.axis_index("s") * _NC + jax.lax.axis_index("c")
    f0 = wid * _FPT
    for f in range(_FPT):
        pltpu.sync_copy(xwt_hbm.at[f0 + f], xws[f])

        @plsc.parallel_loop(0, _N_PAD, step=_L, unroll=8)
        def _(j, _u=uts[f]):
            _u[pl.ds(j, _L)] = jnp.zeros((_L,), jnp.float32)

    def start(k, b):
        srcb, dstb, alb = bufs[b]
        cb = k * _EC
        pltpu.async_copy(src_hbm.at[pl.ds(cb, _EC)], srcb, sems[b])
        pltpu.async_copy(dst_hbm.at[pl.ds(cb, _EC)], dstb, sems[b])
        pltpu.async_copy(al_hbm.at[pl.ds(cb, _EC)], alb, sems[b])

    def wait(b):
        srcb, dstb, alb = bufs[b]
        pltpu.make_async_copy(src_hbm.at[pl.ds(0, _EC)], srcb, sems[b]).wait()
        pltpu.make_async_copy(dst_hbm.at[pl.ds(0, _EC)], dstb, sems[b]).wait()
        pltpu.make_async_copy(al_hbm.at[pl.ds(0, _EC)], alb, sems[b]).wait()

    def compute(b):
        srcb, dstb, alb = bufs[b]

        @plsc.parallel_loop(0, _EC, step=_L, unroll=4)
        def _(i):
            s16 = srcb[pl.ds(i, _L)]
            d16 = dstb[pl.ds(i, _L)]
            a16 = alb[pl.ds(i, _L)]
            for f in range(_FPT):
                g = plsc.load_gather(xws[f], [s16])
                plsc.addupdate_scatter(uts[f], [d16], a16 * g)

    start(0, 0)
    start(1, 1)

    @pl.loop(0, _NCH, step=2)
    def _(k):
        wait(0)
        compute(0)

        @pl.when(k + 2 < _NCH)
        def _():
            start(k + 2, 0)

        wait(1)
        compute(1)

        @pl.when(k + 3 < _NCH)
        def _():
            start(k + 3, 1)

    for f in range(_FPT):
        pltpu.sync_copy(uts[f], ut_hbm.at[f0 + f])


def _layer(src, dst, xw, asv, adv, amax):
    asp = jnp.concatenate(
        [asv[:, 0], jnp.full((_N_PAD - _N,), -1e30, jnp.float32)])
    adp = jnp.concatenate([adv[:, 0], jnp.zeros((_N_PAD - _N,), jnp.float32)])
    avec = jnp.broadcast_to(amax.reshape(1), (_L,))
    ee, den = _edge_a(src, dst, asp, adp, avec)
    rden = _rden(den)[0]
    alpha = _edge_alpha(dst, ee, rden)
    xwt = jnp.pad(xw.T, ((0, 0), (0, _N_PAD - _N)))
    ut = _edge_b(src, dst, alpha, xwt)
    return ut, alpha[:_EALL]


def kernel(x, edge_index, W1, a_src1, a_dst1, b1, W2, a_src2, a_dst2, b2):
    loop = jnp.arange(_N, dtype=edge_index.dtype)
    src0 = jnp.concatenate([edge_index[0], loop])
    dst0 = jnp.concatenate([edge_index[1], loop])
    ei = jnp.stack([src0, dst0], axis=0)
    padi = jnp.full((_E_PAD - _EALL,), _N, jnp.int32)
    src = jnp.concatenate([src0.astype(jnp.int32), padi])
    dst = jnp.concatenate([dst0.astype(jnp.int32), padi])

    xw1, as1, ad1, am1 = _dense1(x, W1, a_src1, a_dst1)
    ut1, alpha1 = _layer(src, dst, xw1, as1, ad1, am1)

    hpre = jnp.transpose(ut1)[:_N]
    xw2, as2, ad2, am2 = _dense2(hpre, b1, W2, a_src2, a_dst2)
    ut2, alpha2 = _layer(src, dst, xw2, as2, ad2, am2)

    out = _bias(jnp.transpose(ut2)[:_N], b2)
    return (out, ((ei, alpha1), (ei, alpha2)))


# speedup vs baseline: 1.5725x; 1.0321x over previous
import dataclasses
import functools

import jax
import jax.numpy as jnp
from jax.experimental import pallas as pl
from jax.experimental.pallas import tpu as pltpu
from jax.experimental.pallas import tpu_sc as plsc

_N = 10000
_E = 320000
_EALL = _E + _N
_D = 128

_NC = 2
_NS = 16
_NW = _NC * _NS
_L = 16
_ET = 10320
_E_PAD = _ET * _NW
_N_PAD = 10016
_FPT = _D // _NW

_mesh = plsc.VectorSubcoreMesh(core_axis_name="c", subcore_axis_name="s")

_sc_params = pltpu.CompilerParams()
if "needs_layout_passes" in pltpu.CompilerParams.__dataclass_fields__:
    _sc_params = dataclasses.replace(_sc_params, needs_layout_passes=False)


def _dense1_body(x_ref, w_ref, asr_ref, adr_ref, xw_ref, as_ref, ad_ref, am_ref):
    xw = jnp.dot(x_ref[...], w_ref[...], preferred_element_type=jnp.float32)
    xw_ref[...] = xw
    asv = jnp.dot(xw, asr_ref[...], preferred_element_type=jnp.float32)
    adv = jnp.dot(xw, adr_ref[...], preferred_element_type=jnp.float32)
    as_ref[...] = asv
    ad_ref[...] = adv
    am_ref[...] = jnp.max(asv).reshape(1, 1)


def _dense1(x, w, a_src, a_dst):
    return pl.pallas_call(
        _dense1_body,
        out_shape=(
            jax.ShapeDtypeStruct((_N, _D), jnp.float32),
            jax.ShapeDtypeStruct((_N, 1), jnp.float32),
            jax.ShapeDtypeStruct((_N, 1), jnp.float32),
            jax.ShapeDtypeStruct((1, 1), jnp.float32),
        ),
    )(x, w, a_src.reshape(_D, 1), a_dst.reshape(_D, 1))


def _dense2_body(hp_ref, b_ref, w_ref, asr_ref, adr_ref,
                 xw_ref, as_ref, ad_ref, am_ref):
    h = hp_ref[...] + b_ref[...]
    h = jnp.where(h > 0, h, jnp.exp(jnp.minimum(h, 0.0)) - 1.0)
    xw = jnp.dot(h, w_ref[...], preferred_element_type=jnp.float32)
    xw_ref[...] = xw
    asv = jnp.dot(xw, asr_ref[...], preferred_element_type=jnp.float32)
    adv = jnp.dot(xw, adr_ref[...], preferred_element_type=jnp.float32)
    as_ref[...] = asv
    ad_ref[...] = adv
    am_ref[...] = jnp.max(asv).reshape(1, 1)


def _dense2(hp, b, w, a_src, a_dst):
    return pl.pallas_call(
        _dense2_body,
        out_shape=(
            jax.ShapeDtypeStruct((_N, _D), jnp.float32),
            jax.ShapeDtypeStruct((_N, 1), jnp.float32),
            jax.ShapeDtypeStruct((_N, 1), jnp.float32),
            jax.ShapeDtypeStruct((1, 1), jnp.float32),
        ),
    )(hp, b.reshape(1, _D), w, a_src.reshape(_D, 1), a_dst.reshape(_D, 1))


def _rden_body(den_ref, r_ref):
    s = jnp.sum(den_ref[...], axis=0, keepdims=True)
    r_ref[...] = 1.0 / (s + 1e-16)


def _rden(den):
    return pl.pallas_call(
        _rden_body,
        out_shape=jax.ShapeDtypeStruct((1, _N_PAD), jnp.float32),
    )(den)


def _bias_body(u_ref, b_ref, o_ref):
    o_ref[...] = u_ref[...] + b_ref[...]


def _bias(u, b):
    return pl.pallas_call(
        _bias_body,
        out_shape=jax.ShapeDtypeStruct((_N, _D), jnp.float32),
    )(u, b.reshape(1, _D))


@functools.partial(
    pl.kernel,
    out_type=(
        jax.ShapeDtypeStruct((_E_PAD,), jnp.float32),
        jax.ShapeDtypeStruct((_NW, _N_PAD), jnp.float32),
        jax.ShapeDtypeStruct((_E_PAD,), jnp.int32),
    ),
    mesh=_mesh,
    compiler_params=_sc_params,
    scratch_types=[
        pltpu.VMEM((_N_PAD,), jnp.float32),
        pltpu.VMEM((_N_PAD,), jnp.float32),
        pltpu.VMEM((_L,), jnp.float32),
        pltpu.VMEM((_ET,), jnp.int32),
        pltpu.VMEM((_ET,), jnp.int32),
        pltpu.VMEM((_ET,), jnp.float32),
        pltpu.VMEM((_N_PAD,), jnp.float32),
        pltpu.VMEM((_ET,), jnp.int32),
    ],
)
def _edge_a(src_hbm, dst_hbm, as_hbm, ad_hbm, av_hbm, ee_hbm, den_hbm, pk_hbm,
            asl, adl, avl, srcb, dstb, eeb, denl, pkb):
    wid = jax.lax.axis_index("s") * _NC + jax.lax.axis_index("c")
    base = wid * _ET
    pltpu.sync_copy(as_hbm, asl)
    pltpu.sync_copy(ad_hbm, adl)
    pltpu.sync_copy(av_hbm, avl)
    pltpu.sync_copy(src_hbm.at[pl.ds(base, _ET)], srcb)
    pltpu.sync_copy(dst_hbm.at[pl.ds(base, _ET)], dstb)

    @plsc.parallel_loop(0, _N_PAD, step=_L, unroll=8)
    def _(j):
        denl[pl.ds(j, _L)] = jnp.zeros((_L,), jnp.float32)

    av = avl[...]

    @plsc.parallel_loop(0, _ET, step=_L, unroll=4)
    def _(i):
        s16 = srcb[pl.ds(i, _L)]
        d16 = dstb[pl.ds(i, _L)]
        a = plsc.load_gather(asl, [s16])
        d = plsc.load_gather(adl, [d16])
        z = a + d
        e = jnp.where(z >= 0, z, 0.2 * z)
        zm = av + d
        m = jnp.where(zm >= 0, zm, 0.2 * zm)
        ee = jnp.exp(e - m)
        eeb[pl.ds(i, _L)] = ee
        pkb[pl.ds(i, _L)] = jnp.bitwise_or(jnp.left_shift(s16, 14), d16)
        plsc.addupdate_scatter(denl, [d16], ee)

    pltpu.sync_copy(eeb, ee_hbm.at[pl.ds(base, _ET)])
    pltpu.sync_copy(pkb, pk_hbm.at[pl.ds(base, _ET)])
    pltpu.sync_copy(denl, den_hbm.at[wid])


@functools.partial(
    pl.kernel,
    out_type=jax.ShapeDtypeStruct((_E_PAD,), jnp.float32),
    mesh=_mesh,
    compiler_params=_sc_params,
    scratch_types=[
        pltpu.VMEM((_N_PAD,), jnp.float32),
        pltpu.VMEM((_ET,), jnp.int32),
        pltpu.VMEM((_ET,), jnp.float32),
    ],
)
def _edge_alpha(dst_hbm, ee_hbm, rd_hbm, al_hbm, rdl, dstb, eeb):
    wid = jax.lax.axis_index("s") * _NC + jax.lax.axis_index("c")
    base = wid * _ET
    pltpu.sync_copy(rd_hbm, rdl)
    pltpu.sync_copy(dst_hbm.at[pl.ds(base, _ET)], dstb)
    pltpu.sync_copy(ee_hbm.at[pl.ds(base, _ET)], eeb)

    @plsc.parallel_loop(0, _ET, step=_L, unroll=4)
    def _(i):
        d16 = dstb[pl.ds(i, _L)]
        r16 = plsc.load_gather(rdl, [d16])
        eeb[pl.ds(i, _L)] = eeb[pl.ds(i, _L)] * r16

    pltpu.sync_copy(eeb, al_hbm.at[pl.ds(base, _ET)])


_EC = 5504
_NCH = _E_PAD // _EC


@functools.partial(
    pl.kernel,
    out_type=jax.ShapeDtypeStruct((_D, _N_PAD), jnp.float32),
    mesh=_mesh,
    compiler_params=_sc_params,
    scratch_types=[
        [pltpu.VMEM((_N_PAD,), jnp.float32) for _ in range(_FPT)],
        [pltpu.VMEM((_N_PAD,), jnp.float32) for _ in range(_FPT)],
        [[pltpu.VMEM((_EC,), jnp.int32),
          pltpu.VMEM((_EC,), jnp.float32)] for _ in range(2)],
        [pltpu.SemaphoreType.DMA for _ in range(2)],
    ],
)
def _edge_b(pk_hbm, al_hbm, xwt_hbm, ut_hbm, xws, uts, bufs, sems):
    wid = jax.lax.axis_index("s") * _NC + jax.lax.axis_index("c")
    f0 = wid * _FPT
    for f in range(_FPT):
        pltpu.sync_copy(xwt_hbm.at[f0 + f], xws[f])

        @plsc.parallel_loop(0, _N_PAD, step=_L, unroll=8)
        def _(j, _u=uts[f]):
            _u[pl.ds(j, _L)] = jnp.zeros((_L,), jnp.float32)

    def start(k, b):
        pkb, alb = bufs[b]
        cb = k * _EC
        pltpu.async_copy(pk_hbm.at[pl.ds(cb, _EC)], pkb, sems[b])
        pltpu.async_copy(al_hbm.at[pl.ds(cb, _EC)], alb, sems[b])

    def wait(b):
        pkb, alb = bufs[b]
        pltpu.make_async_copy(pk_hbm.at[pl.ds(0, _EC)], pkb, sems[b]).wait()
        pltpu.make_async_copy(al_hbm.at[pl.ds(0, _EC)], alb, sems[b]).wait()

    def compute(b):
        pkb, alb = bufs[b]

        @plsc.parallel_loop(0, _EC, step=_L, unroll=4)
        def _(i):
            pk16 = pkb[pl.ds(i, _L)]
            s16 = jax.lax.shift_right_logical(pk16, 14)
            d16 = jnp.bitwise_and(pk16, 16383)
            a16 = alb[pl.ds(i, _L)]
            for f in range(_FPT):
                g = plsc.load_gather(xws[f], [s16])
                plsc.addupdate_scatter(uts[f], [d16], a16 * g)

    start(0, 0)
    start(1, 1)

    @pl.loop(0, _NCH, step=2)
    def _(k):
        wait(0)
        compute(0)

        @pl.when(k + 2 < _NCH)
        def _():
            start(k + 2, 0)

        wait(1)
        compute(1)

        @pl.when(k + 3 < _NCH)
        def _():
            start(k + 3, 1)

    for f in range(_FPT):
        pltpu.sync_copy(uts[f], ut_hbm.at[f0 + f])


def _layer(src, dst, xw, asv, adv, amax):
    asp = jnp.concatenate(
        [asv[:, 0], jnp.full((_N_PAD - _N,), -1e30, jnp.float32)])
    adp = jnp.concatenate([adv[:, 0], jnp.zeros((_N_PAD - _N,), jnp.float32)])
    avec = jnp.broadcast_to(amax.reshape(1), (_L,))
    ee, den, pk = _edge_a(src, dst, asp, adp, avec)
    rden = _rden(den)[0]
    alpha = _edge_alpha(dst, ee, rden)
    xwt = jnp.pad(xw.T, ((0, 0), (0, _N_PAD - _N)))
    ut = _edge_b(pk, alpha, xwt)
    return ut, alpha[:_EALL]


def kernel(x, edge_index, W1, a_src1, a_dst1, b1, W2, a_src2, a_dst2, b2):
    loop = jnp.arange(_N, dtype=edge_index.dtype)
    src0 = jnp.concatenate([edge_index[0], loop])
    dst0 = jnp.concatenate([edge_index[1], loop])
    ei = jnp.stack([src0, dst0], axis=0)
    padi = jnp.full((_E_PAD - _EALL,), _N, jnp.int32)
    src = jnp.concatenate([src0.astype(jnp.int32), padi])
    dst = jnp.concatenate([dst0.astype(jnp.int32), padi])

    xw1, as1, ad1, am1 = _dense1(x, W1, a_src1, a_dst1)
    ut1, alpha1 = _layer(src, dst, xw1, as1, ad1, am1)

    hpre = jnp.transpose(ut1)[:_N]
    xw2, as2, ad2, am2 = _dense2(hpre, b1, W2, a_src2, a_dst2)
    ut2, alpha2 = _layer(src, dst, xw2, as2, ad2, am2)

    out = _bias(jnp.transpose(ut2)[:_N], b2)
    return (out, ((ei, alpha1), (ei, alpha2)))


# speedup vs baseline: 1.6055x; 1.0210x over previous
import dataclasses
import functools

import jax
import jax.numpy as jnp
from jax.experimental import pallas as pl
from jax.experimental.pallas import tpu as pltpu
from jax.experimental.pallas import tpu_sc as plsc

_N = 10000
_E = 320000
_EALL = _E + _N
_D = 128

_NC = 2
_NS = 16
_NW = _NC * _NS
_L = 16
_ET = 10320
_E_PAD = _ET * _NW
_N_PAD = 10016
_FPT = _D // _NW

_mesh = plsc.VectorSubcoreMesh(core_axis_name="c", subcore_axis_name="s")

_sc_params = pltpu.CompilerParams()
if "needs_layout_passes" in pltpu.CompilerParams.__dataclass_fields__:
    _sc_params = dataclasses.replace(_sc_params, needs_layout_passes=False)


def _store_dense_outs(xw, asr_ref, adr_ref, xwt_ref, as_ref, ad_ref, am_ref):
    asv = jnp.dot(xw, asr_ref[...], preferred_element_type=jnp.float32)
    adv = jnp.dot(xw, adr_ref[...], preferred_element_type=jnp.float32)
    as_ref[pl.ds(0, _N), :] = asv
    as_ref[pl.ds(_N, _N_PAD - _N), :] = jnp.full(
        (_N_PAD - _N, 1), -1e30, jnp.float32)
    ad_ref[pl.ds(0, _N), :] = adv
    ad_ref[pl.ds(_N, _N_PAD - _N), :] = jnp.zeros((_N_PAD - _N, 1), jnp.float32)
    am_ref[...] = jnp.max(asv).reshape(1, 1)
    xwt_ref[:, pl.ds(0, _N)] = xw.T


_dense_out_types = (
    jax.ShapeDtypeStruct((_D, _N_PAD), jnp.float32),
    jax.ShapeDtypeStruct((_N_PAD, 1), jnp.float32),
    jax.ShapeDtypeStruct((_N_PAD, 1), jnp.float32),
    jax.ShapeDtypeStruct((1, 1), jnp.float32),
)


def _dense1_body(x_ref, w_ref, asr_ref, adr_ref, xwt_ref, as_ref, ad_ref, am_ref):
    xw = jnp.dot(x_ref[...], w_ref[...], preferred_element_type=jnp.float32)
    _store_dense_outs(xw, asr_ref, adr_ref, xwt_ref, as_ref, ad_ref, am_ref)


def _dense1(x, w, a_src, a_dst):
    return pl.pallas_call(
        _dense1_body,
        out_shape=_dense_out_types,
    )(x, w, a_src.reshape(_D, 1), a_dst.reshape(_D, 1))


def _dense2_body(ut_ref, b_ref, w_ref, asr_ref, adr_ref,
                 xwt_ref, as_ref, ad_ref, am_ref):
    h = ut_ref[:, pl.ds(0, _N)].T + b_ref[...]
    h = jnp.where(h > 0, h, jnp.exp(jnp.minimum(h, 0.0)) - 1.0)
    xw = jnp.dot(h, w_ref[...], preferred_element_type=jnp.float32)
    _store_dense_outs(xw, asr_ref, adr_ref, xwt_ref, as_ref, ad_ref, am_ref)


def _dense2(ut, b, w, a_src, a_dst):
    return pl.pallas_call(
        _dense2_body,
        out_shape=_dense_out_types,
    )(ut, b.reshape(1, _D), w, a_src.reshape(_D, 1), a_dst.reshape(_D, 1))


def _rden_body(den_ref, r_ref):
    s = jnp.sum(den_ref[...], axis=0, keepdims=True)
    r_ref[...] = 1.0 / (s + 1e-16)


def _rden(den):
    return pl.pallas_call(
        _rden_body,
        out_shape=jax.ShapeDtypeStruct((1, _N_PAD), jnp.float32),
    )(den)


def _bias_body(u_ref, b_ref, o_ref):
    o_ref[...] = u_ref[:, pl.ds(0, _N)].T + b_ref[...]


def _bias(u, b):
    return pl.pallas_call(
        _bias_body,
        out_shape=jax.ShapeDtypeStruct((_N, _D), jnp.float32),
    )(u, b.reshape(1, _D))


@functools.partial(
    pl.kernel,
    out_type=(
        jax.ShapeDtypeStruct((_E_PAD,), jnp.float32),
        jax.ShapeDtypeStruct((_NW, _N_PAD), jnp.float32),
        jax.ShapeDtypeStruct((_E_PAD,), jnp.int32),
    ),
    mesh=_mesh,
    compiler_params=_sc_params,
    scratch_types=[
        pltpu.VMEM((_N_PAD,), jnp.float32),
        pltpu.VMEM((_N_PAD,), jnp.float32),
        pltpu.VMEM((_L,), jnp.float32),
        pltpu.VMEM((_ET,), jnp.int32),
        pltpu.VMEM((_ET,), jnp.int32),
        pltpu.VMEM((_ET,), jnp.float32),
        pltpu.VMEM((_N_PAD,), jnp.float32),
        pltpu.VMEM((_ET,), jnp.int32),
    ],
)
def _edge_a(src_hbm, dst_hbm, as_hbm, ad_hbm, av_hbm, ee_hbm, den_hbm, pk_hbm,
            asl, adl, avl, srcb, dstb, eeb, denl, pkb):
    wid = jax.lax.axis_index("s") * _NC + jax.lax.axis_index("c")
    base = wid * _ET
    pltpu.sync_copy(as_hbm, asl)
    pltpu.sync_copy(ad_hbm, adl)
    pltpu.sync_copy(av_hbm, avl)
    pltpu.sync_copy(src_hbm.at[pl.ds(base, _ET)], srcb)
    pltpu.sync_copy(dst_hbm.at[pl.ds(base, _ET)], dstb)

    @plsc.parallel_loop(0, _N_PAD, step=_L, unroll=8)
    def _(j):
        denl[pl.ds(j, _L)] = jnp.zeros((_L,), jnp.float32)

    av = avl[...]

    @plsc.parallel_loop(0, _ET, step=_L, unroll=4)
    def _(i):
        s16 = srcb[pl.ds(i, _L)]
        d16 = dstb[pl.ds(i, _L)]
        a = plsc.load_gather(asl, [s16])
        d = plsc.load_gather(adl, [d16])
        z = a + d
        e = jnp.where(z >= 0, z, 0.2 * z)
        zm = av + d
        m = jnp.where(zm >= 0, zm, 0.2 * zm)
        ee = jnp.exp(e - m)
        eeb[pl.ds(i, _L)] = ee
        pkb[pl.ds(i, _L)] = jnp.bitwise_or(jnp.left_shift(s16, 14), d16)
        plsc.addupdate_scatter(denl, [d16], ee)

    pltpu.sync_copy(eeb, ee_hbm.at[pl.ds(base, _ET)])
    pltpu.sync_copy(pkb, pk_hbm.at[pl.ds(base, _ET)])
    pltpu.sync_copy(denl, den_hbm.at[wid])


@functools.partial(
    pl.kernel,
    out_type=jax.ShapeDtypeStruct((_E_PAD,), jnp.float32),
    mesh=_mesh,
    compiler_params=_sc_params,
    scratch_types=[
        pltpu.VMEM((_N_PAD,), jnp.float32),
        pltpu.VMEM((_ET,), jnp.int32),
        pltpu.VMEM((_ET,), jnp.float32),
    ],
)
def _edge_alpha(dst_hbm, ee_hbm, rd_hbm, al_hbm, rdl, dstb, eeb):
    wid = jax.lax.axis_index("s") * _NC + jax.lax.axis_index("c")
    base = wid * _ET
    pltpu.sync_copy(rd_hbm, rdl)
    pltpu.sync_copy(dst_hbm.at[pl.ds(base, _ET)], dstb)
    pltpu.sync_copy(ee_hbm.at[pl.ds(base, _ET)], eeb)

    @plsc.parallel_loop(0, _ET, step=_L, unroll=4)
    def _(i):
        d16 = dstb[pl.ds(i, _L)]
        r16 = plsc.load_gather(rdl, [d16])
        eeb[pl.ds(i, _L)] = eeb[pl.ds(i, _L)] * r16

    pltpu.sync_copy(eeb, al_hbm.at[pl.ds(base, _ET)])


_EC = 5504
_NCH = _E_PAD // _EC


@functools.partial(
    pl.kernel,
    out_type=jax.ShapeDtypeStruct((_D, _N_PAD), jnp.float32),
    mesh=_mesh,
    compiler_params=_sc_params,
    scratch_types=[
        [pltpu.VMEM((_N_PAD,), jnp.float32) for _ in range(_FPT)],
        [pltpu.VMEM((_N_PAD,), jnp.float32) for _ in range(_FPT)],
        [[pltpu.VMEM((_EC,), jnp.int32),
          pltpu.VMEM((_EC,), jnp.float32)] for _ in range(2)],
        [pltpu.SemaphoreType.DMA for _ in range(2)],
    ],
)
def _edge_b(pk_hbm, al_hbm, xwt_hbm, ut_hbm, xws, uts, bufs, sems):
    wid = jax.lax.axis_index("s") * _NC + jax.lax.axis_index("c")
    f0 = wid * _FPT
    for f in range(_FPT):
        pltpu.sync_copy(xwt_hbm.at[f0 + f], xws[f])

        @plsc.parallel_loop(0, _N_PAD, step=_L, unroll=8)
        def _(j, _u=uts[f]):
            _u[pl.ds(j, _L)] = jnp.zeros((_L,), jnp.float32)

    def start(k, b):
        pkb, alb = bufs[b]
        cb = k * _EC
        pltpu.async_copy(pk_hbm.at[pl.ds(cb, _EC)], pkb, sems[b])
        pltpu.async_copy(al_hbm.at[pl.ds(cb, _EC)], alb, sems[b])

    def wait(b):
        pkb, alb = bufs[b]
        pltpu.make_async_copy(pk_hbm.at[pl.ds(0, _EC)], pkb, sems[b]).wait()
        pltpu.make_async_copy(al_hbm.at[pl.ds(0, _EC)], alb, sems[b]).wait()

    def compute(b):
        pkb, alb = bufs[b]

        @plsc.parallel_loop(0, _EC, step=_L, unroll=4)
        def _(i):
            pk16 = pkb[pl.ds(i, _L)]
            s16 = jax.lax.shift_right_logical(pk16, 14)
            d16 = jnp.bitwise_and(pk16, 16383)
            a16 = alb[pl.ds(i, _L)]
            for f in range(_FPT):
                g = plsc.load_gather(xws[f], [s16])
                plsc.addupdate_scatter(uts[f], [d16], a16 * g)

    start(0, 0)
    start(1, 1)

    @pl.loop(0, _NCH, step=2)
    def _(k):
        wait(0)
        compute(0)

        @pl.when(k + 2 < _NCH)
        def _():
            start(k + 2, 0)

        wait(1)
        compute(1)

        @pl.when(k + 3 < _NCH)
        def _():
            start(k + 3, 1)

    for f in range(_FPT):
        pltpu.sync_copy(uts[f], ut_hbm.at[f0 + f])


def _layer(src, dst, xwt, asv, adv, amax):
    avec = jnp.broadcast_to(amax.reshape(1), (_L,))
    ee, den, pk = _edge_a(src, dst, asv[:, 0], adv[:, 0], avec)
    rden = _rden(den)[0]
    alpha = _edge_alpha(dst, ee, rden)
    ut = _edge_b(pk, alpha, xwt)
    return ut, alpha[:_EALL]


def kernel(x, edge_index, W1, a_src1, a_dst1, b1, W2, a_src2, a_dst2, b2):
    loop = jnp.arange(_N, dtype=edge_index.dtype)
    src0 = jnp.concatenate([edge_index[0], loop])
    dst0 = jnp.concatenate([edge_index[1], loop])
    ei = jnp.stack([src0, dst0], axis=0)
    padi = jnp.full((_E_PAD - _EALL,), _N, jnp.int32)
    src = jnp.concatenate([src0.astype(jnp.int32), padi])
    dst = jnp.concatenate([dst0.astype(jnp.int32), padi])

    xwt1, as1, ad1, am1 = _dense1(x, W1, a_src1, a_dst1)
    ut1, alpha1 = _layer(src, dst, xwt1, as1, ad1, am1)

    xwt2, as2, ad2, am2 = _dense2(ut1, b1, W2, a_src2, a_dst2)
    ut2, alpha2 = _layer(src, dst, xwt2, as2, ad2, am2)

    out = _bias(ut2, b2)
    return (out, ((ei, alpha1), (ei, alpha2)))


# speedup vs baseline: 1.6219x; 1.0102x over previous
import dataclasses
import functools

import jax
import jax.numpy as jnp
from jax.experimental import pallas as pl
from jax.experimental.pallas import tpu as pltpu
from jax.experimental.pallas import tpu_sc as plsc

_N = 10000
_E = 320000
_EALL = _E + _N
_D = 128

_NC = 2
_NS = 16
_NW = _NC * _NS
_L = 16
_ET = 10320
_E_PAD = _ET * _NW
_N_PAD = 10016
_FPT = _D // _NW

_mesh = plsc.VectorSubcoreMesh(core_axis_name="c", subcore_axis_name="s")

_sc_params = pltpu.CompilerParams()
if "needs_layout_passes" in pltpu.CompilerParams.__dataclass_fields__:
    _sc_params = dataclasses.replace(_sc_params, needs_layout_passes=False)


def _store_dense_outs(xw, asr_ref, adr_ref, xwt_ref, as_ref, ad_ref, am_ref):
    asv = jnp.dot(xw, asr_ref[...], preferred_element_type=jnp.float32)
    adv = jnp.dot(xw, adr_ref[...], preferred_element_type=jnp.float32)
    as_ref[pl.ds(0, _N), :] = asv
    as_ref[pl.ds(_N, _N_PAD - _N), :] = jnp.full(
        (_N_PAD - _N, 1), -1e30, jnp.float32)
    ad_ref[pl.ds(0, _N), :] = adv
    ad_ref[pl.ds(_N, _N_PAD - _N), :] = jnp.zeros((_N_PAD - _N, 1), jnp.float32)
    am_ref[...] = jnp.max(asv).reshape(1, 1)
    xwt_ref[:, pl.ds(0, _N)] = xw.T


_dense_out_types = (
    jax.ShapeDtypeStruct((_D, _N_PAD), jnp.float32),
    jax.ShapeDtypeStruct((_N_PAD, 1), jnp.float32),
    jax.ShapeDtypeStruct((_N_PAD, 1), jnp.float32),
    jax.ShapeDtypeStruct((1, 1), jnp.float32),
)


def _dense1_body(x_ref, w_ref, asr_ref, adr_ref, xwt_ref, as_ref, ad_ref, am_ref):
    xw = jnp.dot(x_ref[...], w_ref[...], preferred_element_type=jnp.float32)
    _store_dense_outs(xw, asr_ref, adr_ref, xwt_ref, as_ref, ad_ref, am_ref)


def _dense1(x, w, a_src, a_dst):
    return pl.pallas_call(
        _dense1_body,
        out_shape=_dense_out_types,
    )(x, w, a_src.reshape(_D, 1), a_dst.reshape(_D, 1))


def _dense2_body(ut_ref, b_ref, w_ref, asr_ref, adr_ref,
                 xwt_ref, as_ref, ad_ref, am_ref):
    h = ut_ref[:, pl.ds(0, _N)].T + b_ref[...]
    h = jnp.where(h > 0, h, jnp.exp(jnp.minimum(h, 0.0)) - 1.0)
    xw = jnp.dot(h, w_ref[...], preferred_element_type=jnp.float32)
    _store_dense_outs(xw, asr_ref, adr_ref, xwt_ref, as_ref, ad_ref, am_ref)


def _dense2(ut, b, w, a_src, a_dst):
    return pl.pallas_call(
        _dense2_body,
        out_shape=_dense_out_types,
    )(ut, b.reshape(1, _D), w, a_src.reshape(_D, 1), a_dst.reshape(_D, 1))


def _rden_body(den_ref, r_ref):
    s = jnp.sum(den_ref[...], axis=0, keepdims=True)
    r_ref[...] = 1.0 / (s + 1e-16)


def _rden(den):
    return pl.pallas_call(
        _rden_body,
        out_shape=jax.ShapeDtypeStruct((1, _N_PAD), jnp.float32),
    )(den)


def _bias_body(u_ref, b_ref, o_ref):
    o_ref[...] = u_ref[:, pl.ds(0, _N)].T + b_ref[...]


def _bias(u, b):
    return pl.pallas_call(
        _bias_body,
        out_shape=jax.ShapeDtypeStruct((_N, _D), jnp.float32),
    )(u, b.reshape(1, _D))


@functools.partial(
    pl.kernel,
    out_type=(
        jax.ShapeDtypeStruct((_E_PAD,), jnp.float32),
        jax.ShapeDtypeStruct((_NW, _N_PAD), jnp.float32),
        jax.ShapeDtypeStruct((_E_PAD,), jnp.int32),
    ),
    mesh=_mesh,
    compiler_params=_sc_params,
    scratch_types=[
        pltpu.VMEM((_N_PAD,), jnp.float32),
        pltpu.VMEM((_N_PAD,), jnp.float32),
        pltpu.VMEM((_L,), jnp.float32),
        pltpu.VMEM((_ET,), jnp.int32),
        pltpu.VMEM((_ET,), jnp.int32),
        pltpu.VMEM((_ET,), jnp.float32),
        pltpu.VMEM((_N_PAD,), jnp.float32),
        pltpu.VMEM((_ET,), jnp.int32),
        pltpu.SemaphoreType.DMA,
    ],
)
def _edge_a(src_hbm, dst_hbm, as_hbm, ad_hbm, av_hbm, ee_hbm, den_hbm, pk_hbm,
            asl, adl, avl, srcb, dstb, eeb, denl, pkb, sem):
    wid = jax.lax.axis_index("s") * _NC + jax.lax.axis_index("c")
    base = wid * _ET
    pltpu.async_copy(as_hbm, asl, sem)
    pltpu.async_copy(ad_hbm, adl, sem)
    pltpu.async_copy(av_hbm, avl, sem)
    pltpu.async_copy(src_hbm.at[pl.ds(base, _ET)], srcb, sem)
    pltpu.async_copy(dst_hbm.at[pl.ds(base, _ET)], dstb, sem)

    @plsc.parallel_loop(0, _N_PAD, step=_L, unroll=8)
    def _(j):
        denl[pl.ds(j, _L)] = jnp.zeros((_L,), jnp.float32)

    pltpu.make_async_copy(as_hbm, asl, sem).wait()
    pltpu.make_async_copy(ad_hbm, adl, sem).wait()
    pltpu.make_async_copy(av_hbm, avl, sem).wait()
    pltpu.make_async_copy(src_hbm.at[pl.ds(base, _ET)], srcb, sem).wait()
    pltpu.make_async_copy(dst_hbm.at[pl.ds(base, _ET)], dstb, sem).wait()

    av = avl[...]

    @plsc.parallel_loop(0, _ET, step=_L, unroll=4)
    def _(i):
        s16 = srcb[pl.ds(i, _L)]
        d16 = dstb[pl.ds(i, _L)]
        a = plsc.load_gather(asl, [s16])
        d = plsc.load_gather(adl, [d16])
        z = a + d
        e = jnp.where(z >= 0, z, 0.2 * z)
        zm = av + d
        m = jnp.where(zm >= 0, zm, 0.2 * zm)
        ee = jnp.exp(e - m)
        eeb[pl.ds(i, _L)] = ee
        pkb[pl.ds(i, _L)] = jnp.bitwise_or(jnp.left_shift(s16, 14), d16)
        plsc.addupdate_scatter(denl, [d16], ee)

    pltpu.sync_copy(eeb, ee_hbm.at[pl.ds(base, _ET)])
    pltpu.sync_copy(pkb, pk_hbm.at[pl.ds(base, _ET)])
    pltpu.sync_copy(denl, den_hbm.at[wid])


@functools.partial(
    pl.kernel,
    out_type=jax.ShapeDtypeStruct((_E_PAD,), jnp.float32),
    mesh=_mesh,
    compiler_params=_sc_params,
    scratch_types=[
        pltpu.VMEM((_N_PAD,), jnp.float32),
        pltpu.VMEM((_ET,), jnp.int32),
        pltpu.VMEM((_ET,), jnp.float32),
    ],
)
def _edge_alpha(dst_hbm, ee_hbm, rd_hbm, al_hbm, rdl, dstb, eeb):
    wid = jax.lax.axis_index("s") * _NC + jax.lax.axis_index("c")
    base = wid * _ET
    pltpu.sync_copy(rd_hbm, rdl)
    pltpu.sync_copy(dst_hbm.at[pl.ds(base, _ET)], dstb)
    pltpu.sync_copy(ee_hbm.at[pl.ds(base, _ET)], eeb)

    @plsc.parallel_loop(0, _ET, step=_L, unroll=4)
    def _(i):
        d16 = dstb[pl.ds(i, _L)]
        r16 = plsc.load_gather(rdl, [d16])
        eeb[pl.ds(i, _L)] = eeb[pl.ds(i, _L)] * r16

    pltpu.sync_copy(eeb, al_hbm.at[pl.ds(base, _ET)])


_EC = 5504
_NCH = _E_PAD // _EC


@functools.partial(
    pl.kernel,
    out_type=jax.ShapeDtypeStruct((_D, _N_PAD), jnp.float32),
    mesh=_mesh,
    compiler_params=_sc_params,
    scratch_types=[
        [pltpu.VMEM((_N_PAD,), jnp.float32) for _ in range(_FPT)],
        [pltpu.VMEM((_N_PAD,), jnp.float32) for _ in range(_FPT)],
        [[pltpu.VMEM((_EC,), jnp.int32),
          pltpu.VMEM((_EC,), jnp.float32)] for _ in range(2)],
        [pltpu.SemaphoreType.DMA for _ in range(2)],
    ],
)
def _edge_b(pk_hbm, al_hbm, xwt_hbm, ut_hbm, xws, uts, bufs, sems):
    wid = jax.lax.axis_index("s") * _NC + jax.lax.axis_index("c")
    f0 = wid * _FPT
    for f in range(_FPT):
        pltpu.sync_copy(xwt_hbm.at[f0 + f], xws[f])

        @plsc.parallel_loop(0, _N_PAD, step=_L, unroll=8)
        def _(j, _u=uts[f]):
            _u[pl.ds(j, _L)] = jnp.zeros((_L,), jnp.float32)

    def start(k, b):
        pkb, alb = bufs[b]
        cb = k * _EC
        pltpu.async_copy(pk_hbm.at[pl.ds(cb, _EC)], pkb, sems[b])
        pltpu.async_copy(al_hbm.at[pl.ds(cb, _EC)], alb, sems[b])

    def wait(b):
        pkb, alb = bufs[b]
        pltpu.make_async_copy(pk_hbm.at[pl.ds(0, _EC)], pkb, sems[b]).wait()
        pltpu.make_async_copy(al_hbm.at[pl.ds(0, _EC)], alb, sems[b]).wait()

    def compute(b):
        pkb, alb = bufs[b]

        @plsc.parallel_loop(0, _EC, step=_L, unroll=4)
        def _(i):
            pk16 = pkb[pl.ds(i, _L)]
            s16 = jax.lax.shift_right_logical(pk16, 14)
            d16 = jnp.bitwise_and(pk16, 16383)
            a16 = alb[pl.ds(i, _L)]
            for f in range(_FPT):
                g = plsc.load_gather(xws[f], [s16])
                plsc.addupdate_scatter(uts[f], [d16], a16 * g)

    start(0, 0)
    start(1, 1)

    @pl.loop(0, _NCH, step=2)
    def _(k):
        wait(0)
        compute(0)

        @pl.when(k + 2 < _NCH)
        def _():
            start(k + 2, 0)

        wait(1)
        compute(1)

        @pl.when(k + 3 < _NCH)
        def _():
            start(k + 3, 1)

    for f in range(_FPT):
        pltpu.sync_copy(uts[f], ut_hbm.at[f0 + f])


def _layer(src, dst, xwt, asv, adv, amax):
    avec = jnp.broadcast_to(amax.reshape(1), (_L,))
    ee, den, pk = _edge_a(src, dst, asv[:, 0], adv[:, 0], avec)
    rden = _rden(den)[0]
    alpha = _edge_alpha(dst, ee, rden)
    ut = _edge_b(pk, alpha, xwt)
    return ut, alpha[:_EALL]


def kernel(x, edge_index, W1, a_src1, a_dst1, b1, W2, a_src2, a_dst2, b2):
    loop = jnp.arange(_N, dtype=edge_index.dtype)
    src0 = jnp.concatenate([edge_index[0], loop])
    dst0 = jnp.concatenate([edge_index[1], loop])
    ei = jnp.stack([src0, dst0], axis=0)
    padi = jnp.full((_E_PAD - _EALL,), _N, jnp.int32)
    src = jnp.concatenate([src0.astype(jnp.int32), padi])
    dst = jnp.concatenate([dst0.astype(jnp.int32), padi])

    xwt1, as1, ad1, am1 = _dense1(x, W1, a_src1, a_dst1)
    ut1, alpha1 = _layer(src, dst, xwt1, as1, ad1, am1)

    xwt2, as2, ad2, am2 = _dense2(ut1, b1, W2, a_src2, a_dst2)
    ut2, alpha2 = _layer(src, dst, xwt2, as2, ad2, am2)

    out = _bias(ut2, b2)
    return (out, ((ei, alpha1), (ei, alpha2)))
